# Initial kernel scaffold; baseline (speedup 1.0000x reference)
import jax
import jax.numpy as jnp
from jax.experimental import pallas as pl


def kernel(body_pose_feature, garment_joint_feature, garment_joint_xyz, smpl_cano_verts, skinning_weights, aabb_min, aabb_max, W_bp, b_bp, Wc1, bc1, Wc2, bc2, Wc3, bc3, W_rot, b_rot, W_trans, b_trans):
    raise NotImplementedError("write your pallas kernel here")


# speedup vs baseline: 1.1243x; 1.1243x over previous
import functools

import jax
import jax.numpy as jnp
from jax import lax
from jax.experimental import pallas as pl
from jax.experimental.pallas import tpu as pltpu
from jax.experimental.pallas import tpu_sc as plsc

N_PTS = 100000
N_PAD = 100352
NV = 6890
NV_PAD = 6912
KNN_VT = 512
KNN_VPAD = 7168
KNN_BN = 1024
MLP_BN = 2048
D_P = 128
D_M = 64
D_T = 128
BIG = 3.0e38

_SC_WORKERS = 32
_SC_BPW = N_PAD // _SC_WORKERS
_SC_CHUNKS = 4
_SC_CHUNK = _SC_BPW // _SC_CHUNKS


def _prep_body(sw_ref, bpf_ref, wbp1_ref, a_ref):
    vb = jnp.dot(sw_ref[...], bpf_ref[...], preferred_element_type=jnp.float32)
    a_ref[...] = jnp.dot(vb, wbp1_ref[...], preferred_element_type=jnp.float32)


def _knn_body(x_ref, vt_ref, o_ref):
    x = x_ref[...]
    x2 = jnp.sum(x * x, axis=1, keepdims=True)
    best_d = jnp.full((KNN_BN, 1), BIG, jnp.float32)
    best_i = jnp.zeros((KNN_BN, 1), jnp.int32)
    for t in range(KNN_VPAD // KNN_VT):
        vt = vt_ref[:, t * KNN_VT:(t + 1) * KNN_VT]
        v2 = jnp.sum(vt * vt, axis=0, keepdims=True)
        dot = jnp.dot(x, vt, preferred_element_type=jnp.float32)
        d2 = (x2 - 2.0 * dot) + v2
        gidx = lax.broadcasted_iota(jnp.int32, (KNN_BN, KNN_VT), 1) + t * KNN_VT
        d2 = jnp.where(gidx < NV, d2, BIG)
        tmin = jnp.min(d2, axis=1, keepdims=True)
        targ = jnp.min(jnp.where(d2 <= tmin, gidx, jnp.int32(2**30)),
                       axis=1, keepdims=True)
        upd = tmin < best_d
        best_i = jnp.where(upd, targ, best_i)
        best_d = jnp.where(upd, tmin, best_d)
    o_ref[...] = best_i


def _sc_gather(table, idx):
    mesh = plsc.VectorSubcoreMesh(core_axis_name="c", subcore_axis_name="s")

    @functools.partial(
        pl.kernel, mesh=mesh,
        out_type=jax.ShapeDtypeStruct((N_PAD, D_T), jnp.float32),
        scratch_types=[
            pltpu.VMEM((_SC_CHUNK,), jnp.int32),
            pltpu.VMEM((_SC_CHUNK, D_T), jnp.float32),
            pltpu.SemaphoreType.DMA,
        ],
    )
    def gk(table_hbm, idx_hbm, out_hbm, idx_v, rows_v, sem):
        wid = lax.axis_index("s") * 2 + lax.axis_index("c")
        base = wid * _SC_BPW
        for c in range(_SC_CHUNKS):
            off = base + c * _SC_CHUNK
            pltpu.sync_copy(idx_hbm.at[pl.ds(off, _SC_CHUNK)], idx_v)
            pltpu.async_copy(table_hbm.at[idx_v], rows_v, sem).wait()
            pltpu.sync_copy(rows_v, out_hbm.at[pl.ds(off, _SC_CHUNK)])

    return gk(table, idx)


def _mlp_body(x_ref, gjf_ref, ag_ref, wbp2_ref, bbp_ref, mn_ref, mx_ref,
              w1a_ref, w1b_ref, bc1_ref, wc2_ref, bc2_ref, wc3_ref, bc3_ref,
              wh_ref, bh_ref, o_ref):
    x = x_ref[...]
    xn = 2.0 * (x - mn_ref[...]) / (mx_ref[...] - mn_ref[...]) - 1.0
    z = ag_ref[...] + jnp.dot(gjf_ref[...], wbp2_ref[...],
                              preferred_element_type=jnp.float32) + bbp_ref[...]
    gf = jax.nn.gelu(z)
    w1a = w1a_ref[...]
    h = (xn[:, 0:1] * w1a[0:1, :] + xn[:, 1:2] * w1a[1:2, :]
         + xn[:, 2:3] * w1a[2:3, :])
    h = h + jnp.dot(gf, w1b_ref[...], preferred_element_type=jnp.float32)
    h = jnp.maximum(h + bc1_ref[...], 0.0)
    h = jnp.maximum(jnp.dot(h, wc2_ref[...], preferred_element_type=jnp.float32)
                    + bc2_ref[...], 0.0)
    h = jnp.dot(h, wc3_ref[...], preferred_element_type=jnp.float32) + bc3_ref[...]
    rt = jnp.dot(h, wh_ref[...], preferred_element_type=jnp.float32) + bh_ref[...]
    lane = lax.broadcasted_iota(jnp.int32, (MLP_BN, 8), 1)
    rot_m = lane < 4
    nrm = jnp.sqrt(jnp.sum(jnp.where(rot_m, rt * rt, 0.0), axis=1, keepdims=True))
    rt_n = rt / jnp.maximum(nrm, 1e-12)
    o_ref[...] = jnp.where(rot_m, rt_n, rt)


def kernel(body_pose_feature, garment_joint_feature, garment_joint_xyz,
           smpl_cano_verts, skinning_weights, aabb_min, aabb_max,
           W_bp, b_bp, Wc1, bc1, Wc2, bc2, Wc3, bc3,
           W_rot, b_rot, W_trans, b_trans):
    f32 = jnp.float32
    xyz8 = jnp.pad(garment_joint_xyz, ((0, N_PAD - N_PTS), (0, 5)))
    vt8 = jnp.pad(smpl_cano_verts.T, ((0, 5), (0, KNN_VPAD - NV)))
    gjf = jnp.pad(garment_joint_feature, ((0, N_PAD - N_PTS), (0, 0)))
    swp = jnp.pad(skinning_weights, ((0, NV_PAD - NV), (0, 0)))

    table = pl.pallas_call(
        _prep_body,
        out_shape=jax.ShapeDtypeStruct((NV_PAD, D_T), f32),
    )(swp, body_pose_feature, jnp.pad(W_bp[:D_P], ((0, 0), (0, D_T - D_M))))

    pidx = pl.pallas_call(
        _knn_body,
        grid=(N_PAD // KNN_BN,),
        in_specs=[
            pl.BlockSpec((KNN_BN, 8), lambda i: (i, 0)),
            pl.BlockSpec((8, KNN_VPAD), lambda i: (0, 0)),
        ],
        out_specs=pl.BlockSpec((KNN_BN, 1), lambda i: (i, 0)),
        out_shape=jax.ShapeDtypeStruct((N_PAD, 1), jnp.int32),
        compiler_params=pltpu.CompilerParams(
            dimension_semantics=("arbitrary",)),
    )(xyz8, vt8)

    ag = _sc_gather(table, pidx.reshape(-1))

    mn8 = jnp.pad(aabb_min.reshape(1, 3), ((0, 0), (0, 5)))
    mx8 = jnp.pad(aabb_max.reshape(1, 3), ((0, 0), (0, 5)), constant_values=1.0)
    w1a = jnp.pad(Wc1[:3], ((0, 5), (0, 0)))
    w1b = jnp.pad(Wc1[3:], ((0, D_T - D_M), (0, 0)))
    wh = jnp.pad(jnp.concatenate([W_rot, W_trans], axis=1), ((0, 0), (0, 1)))
    bh = jnp.pad(jnp.concatenate([b_rot, b_trans]).reshape(1, 7), ((0, 0), (0, 1)))

    full = lambda r, c: pl.BlockSpec((r, c), lambda i: (0, 0))
    out8 = pl.pallas_call(
        _mlp_body,
        grid=(N_PAD // MLP_BN,),
        in_specs=[
            pl.BlockSpec((MLP_BN, 8), lambda i: (i, 0)),
            pl.BlockSpec((MLP_BN, D_P), lambda i: (i, 0)),
            pl.BlockSpec((MLP_BN, D_T), lambda i: (i, 0)),
            full(D_P, D_T),
            full(1, D_T),
            full(1, 8), full(1, 8),
            full(8, 128), full(D_T, 128), full(1, 128),
            full(128, 128), full(1, 128),
            full(128, D_M), full(1, D_M),
            full(D_M, 8), full(1, 8),
        ],
        out_specs=pl.BlockSpec((MLP_BN, 8), lambda i: (i, 0)),
        out_shape=jax.ShapeDtypeStruct((N_PAD, 8), f32),
        compiler_params=pltpu.CompilerParams(
            dimension_semantics=("arbitrary",)),
    )(xyz8, gjf, ag,
      jnp.pad(W_bp[D_P:], ((0, 0), (0, D_T - D_M))),
      jnp.pad(b_bp.reshape(1, D_M), ((0, 0), (0, D_T - D_M))), mn8, mx8,
      w1a, w1b, bc1.reshape(1, 128), Wc2, bc2.reshape(1, 128),
      Wc3, bc3.reshape(1, D_M), wh, bh)

    rot = out8[:N_PTS, 0:4]
    trans = out8[:N_PTS, 4:7]
    return (rot, trans)


# speedup vs baseline: 1.1640x; 1.0353x over previous
import functools

import jax
import jax.numpy as jnp
from jax import lax
from jax.experimental import pallas as pl
from jax.experimental.pallas import tpu as pltpu
from jax.experimental.pallas import tpu_sc as plsc

N_PTS = 100000
N_PAD = 100352
NV = 6890
NV_PAD = 6912
KNN_VT = 512
KNN_VPAD = 7168
KNN_BN = 1024
MLP_BN = 2048
D_P = 128
D_M = 64
D_T = 128
BIG = 3.0e38
PADV = 1.0e38

_SC_WORKERS = 32
_SC_BPW = N_PAD // _SC_WORKERS
_SC_CHUNKS = 8
_SC_CHUNK = _SC_BPW // _SC_CHUNKS


def _prep_body(sw_ref, bpf_ref, wbp1_ref, vtp_ref, a_ref, v8_ref, v2_ref):
    vb = jnp.dot(sw_ref[...], bpf_ref[...], preferred_element_type=jnp.float32)
    a_ref[...] = jnp.dot(vb, wbp1_ref[...], preferred_element_type=jnp.float32)
    vt = vtp_ref[0:3, :]
    pm = vtp_ref[3:4, :]
    v2_ref[...] = jnp.sum(vt * vt, axis=0, keepdims=True) + pm * PADV
    v8_ref[...] = jnp.concatenate(
        [-2.0 * vt, jnp.zeros((5, KNN_VPAD), jnp.float32)], axis=0)


def _knn_body(x_ref, v8_ref, v2_ref, o_ref):
    x = x_ref[...]
    x2 = jnp.sum(x * x, axis=1, keepdims=True)
    liota = lax.broadcasted_iota(jnp.int32, (KNN_BN, KNN_VT), 1)
    best_d = jnp.full((KNN_BN, 1), BIG, jnp.float32)
    best_i = jnp.zeros((KNN_BN, 1), jnp.int32)
    for t in range(KNN_VPAD // KNN_VT):
        sl = pl.ds(t * KNN_VT, KNN_VT)
        dotm2 = jnp.dot(x, v8_ref[:, sl], preferred_element_type=jnp.float32)
        d2 = (x2 + dotm2) + v2_ref[:, sl]
        tmin = jnp.min(d2, axis=1, keepdims=True)
        targ = jnp.min(jnp.where(d2 <= tmin, liota, jnp.int32(2**30)),
                       axis=1, keepdims=True)
        upd = tmin < best_d
        best_i = jnp.where(upd, targ + t * KNN_VT, best_i)
        best_d = jnp.where(upd, tmin, best_d)
    o_ref[...] = best_i


def _sc_gather(table, idx):
    mesh = plsc.VectorSubcoreMesh(core_axis_name="c", subcore_axis_name="s")

    @functools.partial(
        pl.kernel, mesh=mesh,
        out_type=jax.ShapeDtypeStruct((N_PAD, D_T), jnp.float32),
        scratch_types=[
            pltpu.VMEM((_SC_CHUNK,), jnp.int32),
            pltpu.VMEM((_SC_CHUNK,), jnp.int32),
            pltpu.VMEM((_SC_CHUNK, D_T), jnp.float32),
            pltpu.VMEM((_SC_CHUNK, D_T), jnp.float32),
            pltpu.SemaphoreType.DMA,
            pltpu.SemaphoreType.DMA,
        ],
    )
    def gk(table_hbm, idx_hbm, out_hbm, idx0, idx1, rows0, rows1, sem0, sem1):
        wid = lax.axis_index("s") * 2 + lax.axis_index("c")
        base = wid * _SC_BPW
        idx_b, rows_b, sem_b = (idx0, idx1), (rows0, rows1), (sem0, sem1)

        def start(c):
            b = c % 2
            off = base + c * _SC_CHUNK
            pltpu.sync_copy(idx_hbm.at[pl.ds(off, _SC_CHUNK)], idx_b[b])
            return pltpu.async_copy(table_hbm.at[idx_b[b]], rows_b[b], sem_b[b])

        pending = start(0)
        for c in range(_SC_CHUNKS):
            nxt = start(c + 1) if c + 1 < _SC_CHUNKS else None
            pending.wait()
            pltpu.sync_copy(rows_b[c % 2],
                            out_hbm.at[pl.ds(base + c * _SC_CHUNK, _SC_CHUNK)])
            pending = nxt

    return gk(table, idx)


def _mlp_body(x_ref, gjf_ref, ag_ref, wbp2_ref, bbp_ref, mn_ref, mx_ref,
              w1a_ref, w1b_ref, bc1_ref, wc2_ref, bc2_ref, wc3_ref, bc3_ref,
              wh_ref, bh_ref, o_ref):
    x = x_ref[...]
    xn = 2.0 * (x - mn_ref[...]) / (mx_ref[...] - mn_ref[...]) - 1.0
    z = ag_ref[...] + jnp.dot(gjf_ref[...], wbp2_ref[...],
                              preferred_element_type=jnp.float32) + bbp_ref[...]
    gf = jax.nn.gelu(z)
    w1a = w1a_ref[...]
    h = (xn[:, 0:1] * w1a[0:1, :] + xn[:, 1:2] * w1a[1:2, :]
         + xn[:, 2:3] * w1a[2:3, :])
    h = h + jnp.dot(gf, w1b_ref[...], preferred_element_type=jnp.float32)
    h = jnp.maximum(h + bc1_ref[...], 0.0)
    h = jnp.maximum(jnp.dot(h, wc2_ref[...], preferred_element_type=jnp.float32)
                    + bc2_ref[...], 0.0)
    h = jnp.dot(h, wc3_ref[...], preferred_element_type=jnp.float32) + bc3_ref[...]
    rt = jnp.dot(h, wh_ref[...], preferred_element_type=jnp.float32) + bh_ref[...]
    lane = lax.broadcasted_iota(jnp.int32, (MLP_BN, 8), 1)
    rot_m = lane < 4
    nrm = jnp.sqrt(jnp.sum(jnp.where(rot_m, rt * rt, 0.0), axis=1, keepdims=True))
    rt_n = rt / jnp.maximum(nrm, 1e-12)
    o_ref[...] = jnp.where(rot_m, rt_n, rt)


def kernel(body_pose_feature, garment_joint_feature, garment_joint_xyz,
           smpl_cano_verts, skinning_weights, aabb_min, aabb_max,
           W_bp, b_bp, Wc1, bc1, Wc2, bc2, Wc3, bc3,
           W_rot, b_rot, W_trans, b_trans):
    f32 = jnp.float32
    x8 = jnp.pad(garment_joint_xyz, ((0, N_PAD - N_PTS), (0, 5)))
    vtp = jnp.pad(smpl_cano_verts.T, ((0, 1), (0, KNN_VPAD - NV)))
    vtp = vtp.at[3, NV:].set(1.0)
    gjf = jnp.pad(garment_joint_feature, ((0, N_PAD - N_PTS), (0, 0)))
    swp = jnp.pad(skinning_weights, ((0, NV_PAD - NV), (0, 0)))

    table, v8, v2 = pl.pallas_call(
        _prep_body,
        out_shape=(jax.ShapeDtypeStruct((NV_PAD, D_T), f32),
                   jax.ShapeDtypeStruct((8, KNN_VPAD), f32),
                   jax.ShapeDtypeStruct((1, KNN_VPAD), f32)),
    )(swp, body_pose_feature, jnp.pad(W_bp[:D_P], ((0, 0), (0, D_T - D_M))), vtp)

    pidx = pl.pallas_call(
        _knn_body,
        grid=(N_PAD // KNN_BN,),
        in_specs=[
            pl.BlockSpec((KNN_BN, 8), lambda i: (i, 0)),
            pl.BlockSpec((8, KNN_VPAD), lambda i: (0, 0)),
            pl.BlockSpec((1, KNN_VPAD), lambda i: (0, 0)),
        ],
        out_specs=pl.BlockSpec((KNN_BN, 1), lambda i: (i, 0)),
        out_shape=jax.ShapeDtypeStruct((N_PAD, 1), jnp.int32),
        compiler_params=pltpu.CompilerParams(
            dimension_semantics=("arbitrary",)),
    )(x8, v8, v2)

    ag = _sc_gather(table, pidx.reshape(-1))

    mn8 = jnp.pad(aabb_min.reshape(1, 3), ((0, 0), (0, 5)))
    mx8 = jnp.pad(aabb_max.reshape(1, 3), ((0, 0), (0, 5)), constant_values=1.0)
    w1a = jnp.pad(Wc1[:3], ((0, 5), (0, 0)))
    w1b = jnp.pad(Wc1[3:], ((0, D_T - D_M), (0, 0)))
    wh = jnp.pad(jnp.concatenate([W_rot, W_trans], axis=1), ((0, 0), (0, 1)))
    bh = jnp.pad(jnp.concatenate([b_rot, b_trans]).reshape(1, 7), ((0, 0), (0, 1)))

    full = lambda r, c: pl.BlockSpec((r, c), lambda i: (0, 0))
    out8 = pl.pallas_call(
        _mlp_body,
        grid=(N_PAD // MLP_BN,),
        in_specs=[
            pl.BlockSpec((MLP_BN, 8), lambda i: (i, 0)),
            pl.BlockSpec((MLP_BN, D_P), lambda i: (i, 0)),
            pl.BlockSpec((MLP_BN, D_T), lambda i: (i, 0)),
            full(D_P, D_T),
            full(1, D_T),
            full(1, 8), full(1, 8),
            full(8, 128), full(D_T, 128), full(1, 128),
            full(128, 128), full(1, 128),
            full(128, D_M), full(1, D_M),
            full(D_M, 8), full(1, 8),
        ],
        out_specs=pl.BlockSpec((MLP_BN, 8), lambda i: (i, 0)),
        out_shape=jax.ShapeDtypeStruct((N_PAD, 8), f32),
        compiler_params=pltpu.CompilerParams(
            dimension_semantics=("arbitrary",)),
    )(x8, gjf, ag,
      jnp.pad(W_bp[D_P:], ((0, 0), (0, D_T - D_M))),
      jnp.pad(b_bp.reshape(1, D_M), ((0, 0), (0, D_T - D_M))), mn8, mx8,
      w1a, w1b, bc1.reshape(1, 128), Wc2, bc2.reshape(1, 128),
      Wc3, bc3.reshape(1, D_M), wh, bh)

    rot = out8[:N_PTS, 0:4]
    trans = out8[:N_PTS, 4:7]
    return (rot, trans)


# speedup vs baseline: 1.7456x; 1.4997x over previous
import functools

import jax
import jax.numpy as jnp
from jax import lax
from jax.experimental import pallas as pl
from jax.experimental.pallas import tpu as pltpu
from jax.experimental.pallas import tpu_sc as plsc

N_PTS = 100000
N_PAD = 100352
NV = 6890
NV_PAD = 6912
KNN_VT = 512
KNN_VPAD = 7168
KNN_BN = 1024
MLP_BN = 2048
D_P = 128
D_M = 64
D_T = 128
BIG = 3.0e38
PADV = 1.0e38

_SC_WORKERS = 32
_SC_BPW = N_PAD // _SC_WORKERS
_SC_CHUNKS = 8
_SC_CHUNK = _SC_BPW // _SC_CHUNKS


def _prep_body(sw_ref, bpf_ref, wbp1_ref, vtp_ref, a_ref, v8_ref, v2_ref):
    vb = jnp.dot(sw_ref[...], bpf_ref[...], preferred_element_type=jnp.float32)
    a_ref[...] = jnp.dot(vb, wbp1_ref[...], preferred_element_type=jnp.float32)
    vt = vtp_ref[0:3, :]
    pm = vtp_ref[3:4, :]
    v2_ref[...] = jnp.sum(vt * vt, axis=0, keepdims=True) + pm * PADV
    v8_ref[...] = jnp.concatenate(
        [-2.0 * vt, jnp.zeros((5, KNN_VPAD), jnp.float32)], axis=0)


def _knn_body(x_ref, v8_ref, v2_ref, o_ref):
    x = x_ref[...]
    liota = lax.broadcasted_iota(
        jnp.int32, (KNN_BN, KNN_VT), 1).astype(jnp.float32)
    best_d = jnp.full((KNN_BN, 1), BIG, jnp.float32)
    best_i = jnp.full((KNN_BN, 1), 0.0, jnp.float32)
    for t in range(KNN_VPAD // KNN_VT):
        sl = pl.ds(t * KNN_VT, KNN_VT)
        dotm2 = jnp.dot(x, v8_ref[:, sl], preferred_element_type=jnp.float32)
        d2 = dotm2 + v2_ref[:, sl]
        tmin = jnp.min(d2, axis=1, keepdims=True)
        targ = jnp.min(jnp.where(d2 <= tmin, liota, BIG),
                       axis=1, keepdims=True)
        upd = tmin < best_d
        best_i = jnp.where(upd, targ + float(t * KNN_VT), best_i)
        best_d = jnp.where(upd, tmin, best_d)
    o_ref[...] = best_i.astype(jnp.int32)


def _sc_gather(table, idx):
    mesh = plsc.VectorSubcoreMesh(core_axis_name="c", subcore_axis_name="s")

    @functools.partial(
        pl.kernel, mesh=mesh,
        out_type=jax.ShapeDtypeStruct((N_PAD, D_T), jnp.float32),
        scratch_types=[
            pltpu.VMEM((_SC_CHUNK,), jnp.int32),
            pltpu.VMEM((_SC_CHUNK,), jnp.int32),
            pltpu.VMEM((_SC_CHUNK, D_T), jnp.float32),
            pltpu.VMEM((_SC_CHUNK, D_T), jnp.float32),
            pltpu.SemaphoreType.DMA,
            pltpu.SemaphoreType.DMA,
        ],
    )
    def gk(table_hbm, idx_hbm, out_hbm, idx0, idx1, rows0, rows1, sem0, sem1):
        wid = lax.axis_index("s") * 2 + lax.axis_index("c")
        base = wid * _SC_BPW
        idx_b, rows_b, sem_b = (idx0, idx1), (rows0, rows1), (sem0, sem1)

        def start(c):
            b = c % 2
            off = base + c * _SC_CHUNK
            pltpu.sync_copy(idx_hbm.at[pl.ds(off, _SC_CHUNK)], idx_b[b])
            return pltpu.async_copy(table_hbm.at[idx_b[b]], rows_b[b], sem_b[b])

        pending = start(0)
        for c in range(_SC_CHUNKS):
            nxt = start(c + 1) if c + 1 < _SC_CHUNKS else None
            pending.wait()
            pltpu.sync_copy(rows_b[c % 2],
                            out_hbm.at[pl.ds(base + c * _SC_CHUNK, _SC_CHUNK)])
            pending = nxt

    return gk(table, idx)


def _mlp_body(x_ref, gjf_ref, ag_ref, wbp2_ref, bbp_ref, mn_ref, mx_ref,
              w1a_ref, w1b_ref, bc1_ref, wc2_ref, bc2_ref, wc3_ref, bc3_ref,
              wh_ref, bh_ref, o_ref):
    x = x_ref[...]
    xn = 2.0 * (x - mn_ref[...]) / (mx_ref[...] - mn_ref[...]) - 1.0
    z = ag_ref[...] + jnp.dot(gjf_ref[...], wbp2_ref[...],
                              preferred_element_type=jnp.float32) + bbp_ref[...]
    gf = jax.nn.gelu(z)
    w1a = w1a_ref[...]
    h = (xn[:, 0:1] * w1a[0:1, :] + xn[:, 1:2] * w1a[1:2, :]
         + xn[:, 2:3] * w1a[2:3, :])
    h = h + jnp.dot(gf, w1b_ref[...], preferred_element_type=jnp.float32)
    h = jnp.maximum(h + bc1_ref[...], 0.0)
    h = jnp.maximum(jnp.dot(h, wc2_ref[...], preferred_element_type=jnp.float32)
                    + bc2_ref[...], 0.0)
    h = jnp.dot(h, wc3_ref[...], preferred_element_type=jnp.float32) + bc3_ref[...]
    rt = jnp.dot(h, wh_ref[...], preferred_element_type=jnp.float32) + bh_ref[...]
    lane = lax.broadcasted_iota(jnp.int32, (MLP_BN, 8), 1)
    rot_m = lane < 4
    nrm = jnp.sqrt(jnp.sum(jnp.where(rot_m, rt * rt, 0.0), axis=1, keepdims=True))
    rt_n = rt / jnp.maximum(nrm, 1e-12)
    o_ref[...] = jnp.where(rot_m, rt_n, rt)


def kernel(body_pose_feature, garment_joint_feature, garment_joint_xyz,
           smpl_cano_verts, skinning_weights, aabb_min, aabb_max,
           W_bp, b_bp, Wc1, bc1, Wc2, bc2, Wc3, bc3,
           W_rot, b_rot, W_trans, b_trans):
    f32 = jnp.float32
    x8 = jnp.pad(garment_joint_xyz, ((0, N_PAD - N_PTS), (0, 5)))
    vtp = jnp.pad(smpl_cano_verts.T, ((0, 1), (0, KNN_VPAD - NV)))
    vtp = vtp.at[3, NV:].set(1.0)
    swp = jnp.pad(skinning_weights, ((0, NV_PAD - NV), (0, 0)))

    table, v8, v2 = pl.pallas_call(
        _prep_body,
        out_shape=(jax.ShapeDtypeStruct((NV_PAD, D_T), f32),
                   jax.ShapeDtypeStruct((8, KNN_VPAD), f32),
                   jax.ShapeDtypeStruct((1, KNN_VPAD), f32)),
    )(swp, body_pose_feature, jnp.pad(W_bp[:D_P], ((0, 0), (0, D_T - D_M))), vtp)

    pidx = pl.pallas_call(
        _knn_body,
        grid=(N_PAD // KNN_BN,),
        in_specs=[
            pl.BlockSpec((KNN_BN, 8), lambda i: (i, 0)),
            pl.BlockSpec((8, KNN_VPAD), lambda i: (0, 0)),
            pl.BlockSpec((1, KNN_VPAD), lambda i: (0, 0)),
        ],
        out_specs=pl.BlockSpec((KNN_BN, 1), lambda i: (i, 0)),
        out_shape=jax.ShapeDtypeStruct((N_PAD, 1), jnp.int32),
        compiler_params=pltpu.CompilerParams(
            dimension_semantics=("arbitrary",)),
    )(x8, v8, v2)

    ag = _sc_gather(table, pidx.reshape(-1))

    mn8 = jnp.pad(aabb_min.reshape(1, 3), ((0, 0), (0, 5)))
    mx8 = jnp.pad(aabb_max.reshape(1, 3), ((0, 0), (0, 5)), constant_values=1.0)
    w1a = jnp.pad(Wc1[:3], ((0, 5), (0, 0)))
    w1b = jnp.pad(Wc1[3:], ((0, D_T - D_M), (0, 0)))
    wh = jnp.pad(jnp.concatenate([W_rot, W_trans], axis=1), ((0, 0), (0, 1)))
    bh = jnp.pad(jnp.concatenate([b_rot, b_trans]).reshape(1, 7), ((0, 0), (0, 1)))

    full = lambda r, c: pl.BlockSpec((r, c), lambda i: (0, 0))
    out8 = pl.pallas_call(
        _mlp_body,
        grid=(N_PAD // MLP_BN,),
        in_specs=[
            pl.BlockSpec((MLP_BN, 8), lambda i: (i, 0)),
            pl.BlockSpec((MLP_BN, D_P), lambda i: (i, 0)),
            pl.BlockSpec((MLP_BN, D_T), lambda i: (i, 0)),
            full(D_P, D_T),
            full(1, D_T),
            full(1, 8), full(1, 8),
            full(8, 128), full(D_T, 128), full(1, 128),
            full(128, 128), full(1, 128),
            full(128, D_M), full(1, D_M),
            full(D_M, 8), full(1, 8),
        ],
        out_specs=pl.BlockSpec((MLP_BN, 8), lambda i: (i, 0)),
        out_shape=jax.ShapeDtypeStruct((N_PTS, 8), f32),
        compiler_params=pltpu.CompilerParams(
            dimension_semantics=("arbitrary",)),
    )(x8, garment_joint_feature, ag,
      jnp.pad(W_bp[D_P:], ((0, 0), (0, D_T - D_M))),
      jnp.pad(b_bp.reshape(1, D_M), ((0, 0), (0, D_T - D_M))), mn8, mx8,
      w1a, w1b, bc1.reshape(1, 128), Wc2, bc2.reshape(1, 128),
      Wc3, bc3.reshape(1, D_M), wh, bh)

    rot = out8[:, 0:4]
    trans = out8[:, 4:7]
    return (rot, trans)


# speedup vs baseline: 1.8199x; 1.0426x over previous
import functools

import jax
import jax.numpy as jnp
from jax import lax
from jax.experimental import pallas as pl
from jax.experimental.pallas import tpu as pltpu
from jax.experimental.pallas import tpu_sc as plsc

N_PTS = 100000
N_PAD = 100352
NV = 6890
NV_PAD = 6912
KNN_VT = 512
KNN_VPAD = 7168
KNN_BN = 1024
MLP_BN = 2048
D_P = 128
D_M = 64
D_T = 128
BIG = 3.0e38
PADV = 1.0e38

N_HALF = N_PAD // 2
_SC_WORKERS = 32
_SC_CHUNK = 392


def _prep_body(sw_ref, bpf_ref, wbp1_ref, vtp_ref, a_ref, v8_ref, v2_ref):
    vb = jnp.dot(sw_ref[...], bpf_ref[...], preferred_element_type=jnp.float32)
    a_ref[...] = jnp.dot(vb, wbp1_ref[...], preferred_element_type=jnp.float32)
    vt = vtp_ref[0:3, :]
    pm = vtp_ref[3:4, :]
    v2_ref[...] = jnp.sum(vt * vt, axis=0, keepdims=True) + pm * PADV
    v8_ref[...] = jnp.concatenate(
        [-2.0 * vt, jnp.zeros((5, KNN_VPAD), jnp.float32)], axis=0)


def _knn_body(x_ref, v8_ref, v2_ref, o_ref):
    x = x_ref[...]
    liota = lax.broadcasted_iota(
        jnp.int32, (KNN_BN, KNN_VT), 1).astype(jnp.float32)
    best_d = jnp.full((KNN_BN, 1), BIG, jnp.float32)
    best_i = jnp.full((KNN_BN, 1), 0.0, jnp.float32)
    for t in range(KNN_VPAD // KNN_VT):
        sl = pl.ds(t * KNN_VT, KNN_VT)
        dotm2 = jnp.dot(x, v8_ref[:, sl], preferred_element_type=jnp.float32)
        d2 = dotm2 + v2_ref[:, sl]
        tmin = jnp.min(d2, axis=1, keepdims=True)
        targ = jnp.min(jnp.where(d2 <= tmin, liota, BIG),
                       axis=1, keepdims=True)
        upd = tmin < best_d
        best_i = jnp.where(upd, targ + float(t * KNN_VT), best_i)
        best_d = jnp.where(upd, tmin, best_d)
    o_ref[...] = best_i.astype(jnp.int32)


def _sc_gather(table, idx, n_rows):
    mesh = plsc.VectorSubcoreMesh(core_axis_name="c", subcore_axis_name="s")
    bpw = n_rows // _SC_WORKERS
    n_chunks = bpw // _SC_CHUNK

    @functools.partial(
        pl.kernel, mesh=mesh,
        out_type=jax.ShapeDtypeStruct((n_rows, D_T), jnp.float32),
        scratch_types=[
            pltpu.VMEM((_SC_CHUNK,), jnp.int32),
            pltpu.VMEM((_SC_CHUNK,), jnp.int32),
            pltpu.VMEM((_SC_CHUNK, D_T), jnp.float32),
            pltpu.VMEM((_SC_CHUNK, D_T), jnp.float32),
            pltpu.SemaphoreType.DMA,
            pltpu.SemaphoreType.DMA,
        ],
    )
    def gk(table_hbm, idx_hbm, out_hbm, idx0, idx1, rows0, rows1, sem0, sem1):
        wid = lax.axis_index("s") * 2 + lax.axis_index("c")
        base = wid * bpw
        idx_b, rows_b, sem_b = (idx0, idx1), (rows0, rows1), (sem0, sem1)

        def start(c):
            b = c % 2
            off = base + c * _SC_CHUNK
            pltpu.sync_copy(idx_hbm.at[pl.ds(off, _SC_CHUNK)], idx_b[b])
            return pltpu.async_copy(table_hbm.at[idx_b[b]], rows_b[b], sem_b[b])

        pending = start(0)
        for c in range(n_chunks):
            nxt = start(c + 1) if c + 1 < n_chunks else None
            pending.wait()
            pltpu.sync_copy(rows_b[c % 2],
                            out_hbm.at[pl.ds(base + c * _SC_CHUNK, _SC_CHUNK)])
            pending = nxt

    return gk(table, idx)


def _mlp_body(x_ref, gjf_ref, ag_ref, wbp2_ref, bbp_ref, mn_ref, mx_ref,
              w1a_ref, w1b_ref, bc1_ref, wc2_ref, bc2_ref, wc3_ref, bc3_ref,
              wh_ref, bh_ref, o_ref):
    x = x_ref[...]
    xn = 2.0 * (x - mn_ref[...]) / (mx_ref[...] - mn_ref[...]) - 1.0
    z = ag_ref[...] + jnp.dot(gjf_ref[...], wbp2_ref[...],
                              preferred_element_type=jnp.float32) + bbp_ref[...]
    gf = jax.nn.gelu(z)
    w1a = w1a_ref[...]
    h = (xn[:, 0:1] * w1a[0:1, :] + xn[:, 1:2] * w1a[1:2, :]
         + xn[:, 2:3] * w1a[2:3, :])
    h = h + jnp.dot(gf, w1b_ref[...], preferred_element_type=jnp.float32)
    h = jnp.maximum(h + bc1_ref[...], 0.0)
    h = jnp.maximum(jnp.dot(h, wc2_ref[...], preferred_element_type=jnp.float32)
                    + bc2_ref[...], 0.0)
    h = jnp.dot(h, wc3_ref[...], preferred_element_type=jnp.float32) + bc3_ref[...]
    rt = jnp.dot(h, wh_ref[...], preferred_element_type=jnp.float32) + bh_ref[...]
    lane = lax.broadcasted_iota(jnp.int32, (MLP_BN, 8), 1)
    rot_m = lane < 4
    nrm = jnp.sqrt(jnp.sum(jnp.where(rot_m, rt * rt, 0.0), axis=1, keepdims=True))
    rt_n = rt / jnp.maximum(nrm, 1e-12)
    o_ref[...] = jnp.where(rot_m, rt_n, rt)


def kernel(body_pose_feature, garment_joint_feature, garment_joint_xyz,
           smpl_cano_verts, skinning_weights, aabb_min, aabb_max,
           W_bp, b_bp, Wc1, bc1, Wc2, bc2, Wc3, bc3,
           W_rot, b_rot, W_trans, b_trans):
    f32 = jnp.float32
    x8 = jnp.pad(garment_joint_xyz, ((0, N_PAD - N_PTS), (0, 5)))
    vtp = jnp.pad(smpl_cano_verts.T, ((0, 1), (0, KNN_VPAD - NV)))
    vtp = vtp.at[3, NV:].set(1.0)
    swp = jnp.pad(skinning_weights, ((0, NV_PAD - NV), (0, 0)))

    table, v8, v2 = pl.pallas_call(
        _prep_body,
        out_shape=(jax.ShapeDtypeStruct((NV_PAD, D_T), f32),
                   jax.ShapeDtypeStruct((8, KNN_VPAD), f32),
                   jax.ShapeDtypeStruct((1, KNN_VPAD), f32)),
    )(swp, body_pose_feature, jnp.pad(W_bp[:D_P], ((0, 0), (0, D_T - D_M))), vtp)

    def knn(x8h):
        return pl.pallas_call(
            _knn_body,
            grid=(x8h.shape[0] // KNN_BN,),
            in_specs=[
                pl.BlockSpec((KNN_BN, 8), lambda i: (i, 0)),
                pl.BlockSpec((8, KNN_VPAD), lambda i: (0, 0)),
                pl.BlockSpec((1, KNN_VPAD), lambda i: (0, 0)),
            ],
            out_specs=pl.BlockSpec((KNN_BN, 1), lambda i: (i, 0)),
            out_shape=jax.ShapeDtypeStruct((x8h.shape[0], 1), jnp.int32),
            compiler_params=pltpu.CompilerParams(
                dimension_semantics=("arbitrary",)),
        )(x8h, v8, v2)

    pidx0 = knn(x8[:N_HALF])
    ag0 = _sc_gather(table, pidx0.reshape(-1), N_HALF)
    pidx1 = knn(x8[N_HALF:])
    ag1 = _sc_gather(table, pidx1.reshape(-1), N_HALF)

    mn8 = jnp.pad(aabb_min.reshape(1, 3), ((0, 0), (0, 5)))
    mx8 = jnp.pad(aabb_max.reshape(1, 3), ((0, 0), (0, 5)), constant_values=1.0)
    w1a = jnp.pad(Wc1[:3], ((0, 5), (0, 0)))
    w1b = jnp.pad(Wc1[3:], ((0, D_T - D_M), (0, 0)))
    wh = jnp.pad(jnp.concatenate([W_rot, W_trans], axis=1), ((0, 0), (0, 1)))
    bh = jnp.pad(jnp.concatenate([b_rot, b_trans]).reshape(1, 7), ((0, 0), (0, 1)))

    full = lambda r, c: pl.BlockSpec((r, c), lambda i: (0, 0))
    wbp2p = jnp.pad(W_bp[D_P:], ((0, 0), (0, D_T - D_M)))
    bbpp = jnp.pad(b_bp.reshape(1, D_M), ((0, 0), (0, D_T - D_M)))

    def mlp(x8h, gjfh, agh, n_out):
        return pl.pallas_call(
            _mlp_body,
            grid=(pl.cdiv(n_out, MLP_BN),),
            in_specs=[
                pl.BlockSpec((MLP_BN, 8), lambda i: (i, 0)),
                pl.BlockSpec((MLP_BN, D_P), lambda i: (i, 0)),
                pl.BlockSpec((MLP_BN, D_T), lambda i: (i, 0)),
                full(D_P, D_T),
                full(1, D_T),
                full(1, 8), full(1, 8),
                full(8, 128), full(D_T, 128), full(1, 128),
                full(128, 128), full(1, 128),
                full(128, D_M), full(1, D_M),
                full(D_M, 8), full(1, 8),
            ],
            out_specs=pl.BlockSpec((MLP_BN, 8), lambda i: (i, 0)),
            out_shape=jax.ShapeDtypeStruct((n_out, 8), f32),
            compiler_params=pltpu.CompilerParams(
                dimension_semantics=("arbitrary",)),
        )(x8h, gjfh, agh, wbp2p, bbpp, mn8, mx8,
          w1a, w1b, bc1.reshape(1, 128), Wc2, bc2.reshape(1, 128),
          Wc3, bc3.reshape(1, D_M), wh, bh)

    outa = mlp(x8[:N_HALF], garment_joint_feature[:N_HALF], ag0, N_HALF)
    outb = mlp(x8[N_HALF:], garment_joint_feature[N_HALF:], ag1,
               N_PTS - N_HALF)
    out8 = jnp.concatenate([outa, outb], axis=0)

    rot = out8[:, 0:4]
    trans = out8[:, 4:7]
    return (rot, trans)


# speedup vs baseline: 1.8242x; 1.0023x over previous
import functools

import jax
import jax.numpy as jnp
from jax import lax
from jax.experimental import pallas as pl
from jax.experimental.pallas import tpu as pltpu
from jax.experimental.pallas import tpu_sc as plsc

N_PTS = 100000
N_PAD = 100352
NV = 6890
NV_PAD = 6912
KNN_VT = 1024
KNN_VPAD = 7168
KNN_BN = 1024
MLP_BN = 1024
D_P = 128
D_M = 64
D_T = 128
BIG = 3.0e38
PADV = 1.0e38

N_HALF = N_PAD // 2
_SC_WORKERS = 32
_SC_CHUNK = 392


def _prep_body(sw_ref, bpf_ref, wbp1_ref, vtp_ref, a_ref, v8_ref, v2_ref):
    vb = jnp.dot(sw_ref[...], bpf_ref[...], preferred_element_type=jnp.float32)
    a_ref[...] = jnp.dot(vb, wbp1_ref[...], preferred_element_type=jnp.float32)
    vt = vtp_ref[0:3, :]
    pm = vtp_ref[3:4, :]
    v2_ref[...] = jnp.sum(vt * vt, axis=0, keepdims=True) + pm * PADV
    v8_ref[...] = jnp.concatenate(
        [-2.0 * vt, jnp.zeros((5, KNN_VPAD), jnp.float32)], axis=0)


def _knn_body(x_ref, v8_ref, v2_ref, o_ref):
    x = x_ref[...]
    liota = lax.broadcasted_iota(
        jnp.int32, (KNN_BN, KNN_VT), 1).astype(jnp.float32)
    best_d = jnp.full((KNN_BN, 1), BIG, jnp.float32)
    best_i = jnp.full((KNN_BN, 1), 0.0, jnp.float32)
    for t in range(KNN_VPAD // KNN_VT):
        sl = pl.ds(t * KNN_VT, KNN_VT)
        dotm2 = jnp.dot(x, v8_ref[:, sl], preferred_element_type=jnp.float32)
        d2 = dotm2 + v2_ref[:, sl]
        tmin = jnp.min(d2, axis=1, keepdims=True)
        targ = jnp.min(jnp.where(d2 <= tmin, liota, BIG),
                       axis=1, keepdims=True)
        upd = tmin < best_d
        best_i = jnp.where(upd, targ + float(t * KNN_VT), best_i)
        best_d = jnp.where(upd, tmin, best_d)
    o_ref[...] = best_i.astype(jnp.int32)


def _sc_gather(table, idx, n_rows):
    mesh = plsc.VectorSubcoreMesh(core_axis_name="c", subcore_axis_name="s")
    bpw = n_rows // _SC_WORKERS
    n_chunks = bpw // _SC_CHUNK

    @functools.partial(
        pl.kernel, mesh=mesh,
        out_type=jax.ShapeDtypeStruct((n_rows, D_T), jnp.float32),
        scratch_types=[
            pltpu.VMEM((_SC_CHUNK,), jnp.int32),
            pltpu.VMEM((_SC_CHUNK,), jnp.int32),
            pltpu.VMEM((_SC_CHUNK, D_T), jnp.float32),
            pltpu.VMEM((_SC_CHUNK, D_T), jnp.float32),
            pltpu.SemaphoreType.DMA,
            pltpu.SemaphoreType.DMA,
        ],
    )
    def gk(table_hbm, idx_hbm, out_hbm, idx0, idx1, rows0, rows1, sem0, sem1):
        wid = lax.axis_index("s") * 2 + lax.axis_index("c")
        base = wid * bpw
        idx_b, rows_b, sem_b = (idx0, idx1), (rows0, rows1), (sem0, sem1)

        def start(c):
            b = c % 2
            off = base + c * _SC_CHUNK
            pltpu.sync_copy(idx_hbm.at[pl.ds(off, _SC_CHUNK)], idx_b[b])
            return pltpu.async_copy(table_hbm.at[idx_b[b]], rows_b[b], sem_b[b])

        pending = start(0)
        for c in range(n_chunks):
            nxt = start(c + 1) if c + 1 < n_chunks else None
            pending.wait()
            pltpu.sync_copy(rows_b[c % 2],
                            out_hbm.at[pl.ds(base + c * _SC_CHUNK, _SC_CHUNK)])
            pending = nxt

    return gk(table, idx)


def _mlp_body(x_ref, gjf_ref, ag_ref, wbp2_ref, bbp_ref, mn_ref, mx_ref,
              w1a_ref, w1b_ref, bc1_ref, wc2_ref, bc2_ref, wc3_ref, bc3_ref,
              wh_ref, bh_ref, o_ref):
    x = x_ref[...]
    xn = 2.0 * (x - mn_ref[...]) / (mx_ref[...] - mn_ref[...]) - 1.0
    z = ag_ref[...] + jnp.dot(gjf_ref[...], wbp2_ref[...],
                              preferred_element_type=jnp.float32) + bbp_ref[...]
    gf = jax.nn.gelu(z)
    w1a = w1a_ref[...]
    h = (xn[:, 0:1] * w1a[0:1, :] + xn[:, 1:2] * w1a[1:2, :]
         + xn[:, 2:3] * w1a[2:3, :])
    h = h + jnp.dot(gf, w1b_ref[...], preferred_element_type=jnp.float32)
    h = jnp.maximum(h + bc1_ref[...], 0.0)
    h = jnp.maximum(jnp.dot(h, wc2_ref[...], preferred_element_type=jnp.float32)
                    + bc2_ref[...], 0.0)
    h = jnp.dot(h, wc3_ref[...], preferred_element_type=jnp.float32) + bc3_ref[...]
    rt = jnp.dot(h, wh_ref[...], preferred_element_type=jnp.float32) + bh_ref[...]
    lane = lax.broadcasted_iota(jnp.int32, (MLP_BN, 8), 1)
    rot_m = lane < 4
    nrm = jnp.sqrt(jnp.sum(jnp.where(rot_m, rt * rt, 0.0), axis=1, keepdims=True))
    rt_n = rt / jnp.maximum(nrm, 1e-12)
    o_ref[...] = jnp.where(rot_m, rt_n, rt)


def kernel(body_pose_feature, garment_joint_feature, garment_joint_xyz,
           smpl_cano_verts, skinning_weights, aabb_min, aabb_max,
           W_bp, b_bp, Wc1, bc1, Wc2, bc2, Wc3, bc3,
           W_rot, b_rot, W_trans, b_trans):
    f32 = jnp.float32
    x8 = jnp.pad(garment_joint_xyz, ((0, N_PAD - N_PTS), (0, 5)))
    vtp = jnp.pad(smpl_cano_verts.T, ((0, 1), (0, KNN_VPAD - NV)))
    vtp = vtp.at[3, NV:].set(1.0)
    swp = jnp.pad(skinning_weights, ((0, NV_PAD - NV), (0, 0)))

    table, v8, v2 = pl.pallas_call(
        _prep_body,
        out_shape=(jax.ShapeDtypeStruct((NV_PAD, D_T), f32),
                   jax.ShapeDtypeStruct((8, KNN_VPAD), f32),
                   jax.ShapeDtypeStruct((1, KNN_VPAD), f32)),
    )(swp, body_pose_feature, jnp.pad(W_bp[:D_P], ((0, 0), (0, D_T - D_M))), vtp)

    def knn(half):
        blk0 = half * (N_HALF // KNN_BN)
        return pl.pallas_call(
            _knn_body,
            grid=(N_HALF // KNN_BN,),
            in_specs=[
                pl.BlockSpec((KNN_BN, 8), lambda i: (i + blk0, 0)),
                pl.BlockSpec((8, KNN_VPAD), lambda i: (0, 0)),
                pl.BlockSpec((1, KNN_VPAD), lambda i: (0, 0)),
            ],
            out_specs=pl.BlockSpec((KNN_BN, 1), lambda i: (i, 0)),
            out_shape=jax.ShapeDtypeStruct((N_HALF, 1), jnp.int32),
            compiler_params=pltpu.CompilerParams(
                dimension_semantics=("arbitrary",)),
        )(x8, v8, v2)

    pidx0 = knn(0)
    ag0 = _sc_gather(table, pidx0.reshape(-1), N_HALF)
    pidx1 = knn(1)
    ag1 = _sc_gather(table, pidx1.reshape(-1), N_HALF)

    mn8 = jnp.pad(aabb_min.reshape(1, 3), ((0, 0), (0, 5)))
    mx8 = jnp.pad(aabb_max.reshape(1, 3), ((0, 0), (0, 5)), constant_values=1.0)
    w1a = jnp.pad(Wc1[:3], ((0, 5), (0, 0)))
    w1b = jnp.pad(Wc1[3:], ((0, D_T - D_M), (0, 0)))
    wh = jnp.pad(jnp.concatenate([W_rot, W_trans], axis=1), ((0, 0), (0, 1)))
    bh = jnp.pad(jnp.concatenate([b_rot, b_trans]).reshape(1, 7), ((0, 0), (0, 1)))

    full = lambda r, c: pl.BlockSpec((r, c), lambda i: (0, 0))
    wbp2p = jnp.pad(W_bp[D_P:], ((0, 0), (0, D_T - D_M)))
    bbpp = jnp.pad(b_bp.reshape(1, D_M), ((0, 0), (0, D_T - D_M)))

    def mlp(half, agh, n_out):
        blk0 = half * (N_HALF // MLP_BN)
        return pl.pallas_call(
            _mlp_body,
            grid=(pl.cdiv(n_out, MLP_BN),),
            in_specs=[
                pl.BlockSpec((MLP_BN, 8), lambda i: (i + blk0, 0)),
                pl.BlockSpec((MLP_BN, D_P), lambda i: (i + blk0, 0)),
                pl.BlockSpec((MLP_BN, D_T), lambda i: (i, 0)),
                full(D_P, D_T),
                full(1, D_T),
                full(1, 8), full(1, 8),
                full(8, 128), full(D_T, 128), full(1, 128),
                full(128, 128), full(1, 128),
                full(128, D_M), full(1, D_M),
                full(D_M, 8), full(1, 8),
            ],
            out_specs=pl.BlockSpec((MLP_BN, 8), lambda i: (i, 0)),
            out_shape=jax.ShapeDtypeStruct((n_out, 8), f32),
            compiler_params=pltpu.CompilerParams(
                dimension_semantics=("arbitrary",)),
        )(x8, garment_joint_feature, agh, wbp2p, bbpp, mn8, mx8,
          w1a, w1b, bc1.reshape(1, 128), Wc2, bc2.reshape(1, 128),
          Wc3, bc3.reshape(1, D_M), wh, bh)

    outa = mlp(0, ag0, N_HALF)
    outb = mlp(1, ag1, N_PTS - N_HALF)
    out8 = jnp.concatenate([outa, outb], axis=0)

    rot = out8[:, 0:4]
    trans = out8[:, 4:7]
    return (rot, trans)


# speedup vs baseline: 1.8983x; 1.0406x over previous
import functools

import jax
import jax.numpy as jnp
from jax import lax
from jax.experimental import pallas as pl
from jax.experimental.pallas import tpu as pltpu
from jax.experimental.pallas import tpu_sc as plsc

N_PTS = 100000
N_PAD = 100352
NV = 6890
NV_PAD = 6912
KNN_VT = 1024
KNN_VPAD = 7168
KNN_BN = 1792
MLP_BN = 1792
D_P = 128
D_M = 64
D_T = 128
BIG = 3.0e38
PADV = 1.0e38

N_HALF = N_PAD // 2
_SC_WORKERS = 32
_SC_CHUNK = 392


def _prep_body(sw_ref, bpf_ref, wbp1_ref, vtp_ref, a_ref, v8_ref, v2_ref):
    vb = jnp.dot(sw_ref[...], bpf_ref[...], preferred_element_type=jnp.float32)
    a_ref[...] = jnp.dot(vb, wbp1_ref[...], preferred_element_type=jnp.float32)
    vt = vtp_ref[0:3, :]
    pm = vtp_ref[3:4, :]
    v2_ref[...] = jnp.sum(vt * vt, axis=0, keepdims=True) + pm * PADV
    v8_ref[...] = jnp.concatenate(
        [-2.0 * vt, jnp.zeros((5, KNN_VPAD), jnp.float32)], axis=0)


def _knn_body(x_ref, v8_ref, v2_ref, o_ref):
    x = x_ref[...]
    liota = lax.broadcasted_iota(
        jnp.int32, (KNN_BN, KNN_VT), 1).astype(jnp.float32)
    best_d = jnp.full((KNN_BN, 1), BIG, jnp.float32)
    best_i = jnp.full((KNN_BN, 1), 0.0, jnp.float32)
    for t in range(KNN_VPAD // KNN_VT):
        sl = pl.ds(t * KNN_VT, KNN_VT)
        dotm2 = jnp.dot(x, v8_ref[:, sl], preferred_element_type=jnp.float32)
        d2 = dotm2 + v2_ref[:, sl]
        tmin = jnp.min(d2, axis=1, keepdims=True)
        targ = jnp.min(jnp.where(d2 <= tmin, liota, BIG),
                       axis=1, keepdims=True)
        upd = tmin < best_d
        best_i = jnp.where(upd, targ + float(t * KNN_VT), best_i)
        best_d = jnp.where(upd, tmin, best_d)
    o_ref[...] = best_i.astype(jnp.int32)


def _sc_gather(table, idx, n_rows):
    mesh = plsc.VectorSubcoreMesh(core_axis_name="c", subcore_axis_name="s")
    bpw = n_rows // _SC_WORKERS
    n_chunks = bpw // _SC_CHUNK

    @functools.partial(
        pl.kernel, mesh=mesh,
        out_type=jax.ShapeDtypeStruct((n_rows, D_T), jnp.float32),
        scratch_types=[
            pltpu.VMEM((_SC_CHUNK,), jnp.int32),
            pltpu.VMEM((_SC_CHUNK,), jnp.int32),
            pltpu.VMEM((_SC_CHUNK, D_T), jnp.float32),
            pltpu.VMEM((_SC_CHUNK, D_T), jnp.float32),
            pltpu.SemaphoreType.DMA,
            pltpu.SemaphoreType.DMA,
        ],
    )
    def gk(table_hbm, idx_hbm, out_hbm, idx0, idx1, rows0, rows1, sem0, sem1):
        wid = lax.axis_index("s") * 2 + lax.axis_index("c")
        base = wid * bpw
        idx_b, rows_b, sem_b = (idx0, idx1), (rows0, rows1), (sem0, sem1)

        def start(c):
            b = c % 2
            off = base + c * _SC_CHUNK
            pltpu.sync_copy(idx_hbm.at[pl.ds(off, _SC_CHUNK)], idx_b[b])
            return pltpu.async_copy(table_hbm.at[idx_b[b]], rows_b[b], sem_b[b])

        pending = start(0)
        for c in range(n_chunks):
            nxt = start(c + 1) if c + 1 < n_chunks else None
            pending.wait()
            pltpu.sync_copy(rows_b[c % 2],
                            out_hbm.at[pl.ds(base + c * _SC_CHUNK, _SC_CHUNK)])
            pending = nxt

    return gk(table, idx)


def _mlp_body(x_ref, gjf_ref, ag_ref, wbp2_ref, bbp_ref, mn_ref, mx_ref,
              w1a_ref, w1b_ref, bc1_ref, wc2_ref, bc2_ref, wc3_ref, bc3_ref,
              wh_ref, bh_ref, o_ref):
    x = x_ref[...]
    xn = 2.0 * (x - mn_ref[...]) / (mx_ref[...] - mn_ref[...]) - 1.0
    z = ag_ref[...] + jnp.dot(gjf_ref[...], wbp2_ref[...],
                              preferred_element_type=jnp.float32) + bbp_ref[...]
    gf = jax.nn.gelu(z)
    w1a = w1a_ref[...]
    h = (xn[:, 0:1] * w1a[0:1, :] + xn[:, 1:2] * w1a[1:2, :]
         + xn[:, 2:3] * w1a[2:3, :])
    h = h + jnp.dot(gf, w1b_ref[...], preferred_element_type=jnp.float32)
    h = jnp.maximum(h + bc1_ref[...], 0.0)
    h = jnp.maximum(jnp.dot(h, wc2_ref[...], preferred_element_type=jnp.float32)
                    + bc2_ref[...], 0.0)
    h = jnp.dot(h, wc3_ref[...], preferred_element_type=jnp.float32) + bc3_ref[...]
    rt = jnp.dot(h, wh_ref[...], preferred_element_type=jnp.float32) + bh_ref[...]
    lane = lax.broadcasted_iota(jnp.int32, (MLP_BN, 8), 1)
    rot_m = lane < 4
    nrm = jnp.sqrt(jnp.sum(jnp.where(rot_m, rt * rt, 0.0), axis=1, keepdims=True))
    rt_n = rt / jnp.maximum(nrm, 1e-12)
    o_ref[...] = jnp.where(rot_m, rt_n, rt)


def kernel(body_pose_feature, garment_joint_feature, garment_joint_xyz,
           smpl_cano_verts, skinning_weights, aabb_min, aabb_max,
           W_bp, b_bp, Wc1, bc1, Wc2, bc2, Wc3, bc3,
           W_rot, b_rot, W_trans, b_trans):
    f32 = jnp.float32
    x8 = jnp.pad(garment_joint_xyz, ((0, N_PAD - N_PTS), (0, 5)))
    vtp = jnp.pad(smpl_cano_verts.T, ((0, 1), (0, KNN_VPAD - NV)))
    vtp = vtp.at[3, NV:].set(1.0)
    swp = jnp.pad(skinning_weights, ((0, NV_PAD - NV), (0, 0)))

    table, v8, v2 = pl.pallas_call(
        _prep_body,
        out_shape=(jax.ShapeDtypeStruct((NV_PAD, D_T), f32),
                   jax.ShapeDtypeStruct((8, KNN_VPAD), f32),
                   jax.ShapeDtypeStruct((1, KNN_VPAD), f32)),
    )(swp, body_pose_feature, jnp.pad(W_bp[:D_P], ((0, 0), (0, D_T - D_M))), vtp)

    def knn(half):
        blk0 = half * (N_HALF // KNN_BN)
        return pl.pallas_call(
            _knn_body,
            grid=(N_HALF // KNN_BN,),
            in_specs=[
                pl.BlockSpec((KNN_BN, 8), lambda i: (i + blk0, 0)),
                pl.BlockSpec((8, KNN_VPAD), lambda i: (0, 0)),
                pl.BlockSpec((1, KNN_VPAD), lambda i: (0, 0)),
            ],
            out_specs=pl.BlockSpec((KNN_BN, 1), lambda i: (i, 0)),
            out_shape=jax.ShapeDtypeStruct((N_HALF, 1), jnp.int32),
            compiler_params=pltpu.CompilerParams(
                dimension_semantics=("arbitrary",)),
        )(x8, v8, v2)

    pidx0 = knn(0)
    ag0 = _sc_gather(table, pidx0.reshape(-1), N_HALF)
    pidx1 = knn(1)
    ag1 = _sc_gather(table, pidx1.reshape(-1), N_HALF)

    mn8 = jnp.pad(aabb_min.reshape(1, 3), ((0, 0), (0, 5)))
    mx8 = jnp.pad(aabb_max.reshape(1, 3), ((0, 0), (0, 5)), constant_values=1.0)
    w1a = jnp.pad(Wc1[:3], ((0, 5), (0, 0)))
    w1b = jnp.pad(Wc1[3:], ((0, D_T - D_M), (0, 0)))
    wh = jnp.pad(jnp.concatenate([W_rot, W_trans], axis=1), ((0, 0), (0, 1)))
    bh = jnp.pad(jnp.concatenate([b_rot, b_trans]).reshape(1, 7), ((0, 0), (0, 1)))

    full = lambda r, c: pl.BlockSpec((r, c), lambda i: (0, 0))
    wbp2p = jnp.pad(W_bp[D_P:], ((0, 0), (0, D_T - D_M)))
    bbpp = jnp.pad(b_bp.reshape(1, D_M), ((0, 0), (0, D_T - D_M)))

    def mlp(half, agh, n_out):
        blk0 = half * (N_HALF // MLP_BN)
        return pl.pallas_call(
            _mlp_body,
            grid=(pl.cdiv(n_out, MLP_BN),),
            in_specs=[
                pl.BlockSpec((MLP_BN, 8), lambda i: (i + blk0, 0)),
                pl.BlockSpec((MLP_BN, D_P), lambda i: (i + blk0, 0)),
                pl.BlockSpec((MLP_BN, D_T), lambda i: (i, 0)),
                full(D_P, D_T),
                full(1, D_T),
                full(1, 8), full(1, 8),
                full(8, 128), full(D_T, 128), full(1, 128),
                full(128, 128), full(1, 128),
                full(128, D_M), full(1, D_M),
                full(D_M, 8), full(1, 8),
            ],
            out_specs=pl.BlockSpec((MLP_BN, 8), lambda i: (i, 0)),
            out_shape=jax.ShapeDtypeStruct((n_out, 8), f32),
            compiler_params=pltpu.CompilerParams(
                dimension_semantics=("arbitrary",)),
        )(x8, garment_joint_feature, agh, wbp2p, bbpp, mn8, mx8,
          w1a, w1b, bc1.reshape(1, 128), Wc2, bc2.reshape(1, 128),
          Wc3, bc3.reshape(1, D_M), wh, bh)

    outa = mlp(0, ag0, N_HALF)
    outb = mlp(1, ag1, N_PTS - N_HALF)
    out8 = jnp.concatenate([outa, outb], axis=0)

    rot = out8[:, 0:4]
    trans = out8[:, 4:7]
    return (rot, trans)
